# baseline (speedup 1.0000x reference)
import functools

import jax
import jax.numpy as jnp
from jax import lax
from jax.experimental import pallas as pl
from jax.experimental.pallas import tpu as pltpu
from jax.experimental.pallas import tpu_sc as plsc

_B, _N, _T, _F = 4, 2500, 12, 128
_HID = 128
_LOOKAHEAD = 12
_NTOT = _B * _N
_E = 320000

_NC, _NS = 2, 16
_NTILE = _NC * _NS
_CH = 128
_NCHUNK = 80
_EPAD = _NTILE * _NCHUNK * _CH
_NPAD = 10240
_RPT = _NPAD // _NS
_ZB = 128
_CNTW = 128
_G = 16
_NGRP = _NCHUNK // _G

_mesh = plsc.VectorSubcoreMesh(core_axis_name="c", subcore_axis_name="s")


@functools.partial(
    pl.kernel,
    out_type=jax.ShapeDtypeStruct((_NC, _NPAD, _CNTW), jnp.float32),
    mesh=_mesh,
    scratch_types=[
        pltpu.VMEM((_CH, _CNTW), jnp.float32),
        pltpu.VMEM((_G, _CH), jnp.int32),
        pltpu.VMEM_SHARED((_NPAD, _CNTW), jnp.float32),
        pltpu.SemaphoreType.DMA,
    ],
)
def _sc_count(dst_hbm, out_hbm, buf_v, dig_v, cnt_sh, sem):
    c = lax.axis_index("c")
    s = lax.axis_index("s")
    wid = c * _NS + s

    @pl.loop(0, _CH)
    def _(i):
        for jj in range(_CNTW // 16):
            buf_v[i, pl.ds(jj * 16, 16)] = jnp.zeros((16,), jnp.float32)

    for k in range(_RPT // _ZB):
        pltpu.sync_copy(buf_v, cnt_sh.at[pl.ds(s * _RPT + k * _ZB, _ZB)])

    @pl.loop(0, _CH)
    def _(i):
        for jj in range(_CNTW // 16):
            buf_v[i, pl.ds(jj * 16, 16)] = jnp.full((16,), 1.0, jnp.float32)

    plsc.subcore_barrier()

    for g in range(_NGRP):
        pltpu.sync_copy(dst_hbm.at[wid, pl.ds(g * _G, _G)], dig_v)

        @pl.loop(0, _G, step=4)
        def _(j):
            handles = [
                pltpu.async_copy(buf_v, cnt_sh.at[dig_v.at[j + d]], sem,
                                 add=True)
                for d in range(4)
            ]
            for h in handles:
                h.wait()

    plsc.subcore_barrier()
    pltpu.sync_copy(cnt_sh.at[pl.ds(s * _RPT, _RPT)],
                    out_hbm.at[c, pl.ds(s * _RPT, _RPT)])


@functools.partial(
    pl.kernel,
    out_type=jax.ShapeDtypeStruct((_NC, _NPAD, _F), jnp.float32),
    mesh=_mesh,
    scratch_types=[
        pltpu.VMEM((_CH, _F), jnp.float32),
        pltpu.VMEM((_CH, _F), jnp.float32),
        pltpu.VMEM((_G, _CH), jnp.int32),
        pltpu.VMEM((_G, _CH), jnp.int32),
        pltpu.VMEM_SHARED((_NPAD, _F), jnp.float32),
        pltpu.SemaphoreType.DMA,
        pltpu.SemaphoreType.DMA,
    ],
)
def _sc_scatter(xs_hbm, src_hbm, dst_hbm, out_hbm,
                rows0_v, rows1_v, sig_v, dig_v, acc_sh, sem0, sem1):
    c = lax.axis_index("c")
    s = lax.axis_index("s")
    wid = c * _NS + s

    @pl.loop(0, _ZB)
    def _(i):
        for jj in range(_F // 16):
            rows0_v[i, pl.ds(jj * 16, 16)] = jnp.zeros((16,), jnp.float32)

    for k in range(_RPT // _ZB):
        pltpu.sync_copy(rows0_v, acc_sh.at[pl.ds(s * _RPT + k * _ZB, _ZB)])
    plsc.subcore_barrier()

    for g in range(_NGRP):
        pltpu.sync_copy(src_hbm.at[wid, pl.ds(g * _G, _G)], sig_v)
        pltpu.sync_copy(dst_hbm.at[wid, pl.ds(g * _G, _G)], dig_v)
        pltpu.async_copy(xs_hbm.at[sig_v.at[0]], rows0_v, sem0)

        @pl.loop(0, _G, step=2)
        def _(j):
            pltpu.async_copy(xs_hbm.at[sig_v.at[j + 1]], rows1_v, sem1)
            pltpu.make_async_copy(xs_hbm.at[pl.ds(0, _CH)], rows0_v,
                                  sem0).wait()
            pltpu.sync_copy(rows0_v, acc_sh.at[dig_v.at[j]], add=True)

            @pl.when(j + 2 < _G)
            def _():
                pltpu.async_copy(xs_hbm.at[sig_v.at[j + 2]], rows0_v, sem0)

            pltpu.make_async_copy(xs_hbm.at[pl.ds(0, _CH)], rows1_v,
                                  sem1).wait()
            pltpu.sync_copy(rows1_v, acc_sh.at[dig_v.at[j + 1]], add=True)

    plsc.subcore_barrier()
    pltpu.sync_copy(acc_sh.at[pl.ds(s * _RPT, _RPT)],
                    out_hbm.at[c, pl.ds(s * _RPT, _RPT)])


_ROWS = 1000


def _prep_body(x_ref, cnt_ref, xs_ref):
    deg = 1.0 + cnt_ref[0, :, 0:1] + cnt_ref[1, :, 0:1]
    xs_ref[...] = x_ref[...] * lax.rsqrt(deg)


def _prep_call(x, counts):
    return pl.pallas_call(
        _prep_body,
        grid=(_NTOT // _ROWS,),
        in_specs=[
            pl.BlockSpec((_ROWS, _F), lambda i: (i, 0)),
            pl.BlockSpec((_NC, _ROWS, _CNTW), lambda i: (0, i, 0)),
        ],
        out_specs=pl.BlockSpec((_ROWS, _F), lambda i: (i, 0)),
        out_shape=jax.ShapeDtypeStruct((_NTOT, _F), jnp.float32),
    )(x, counts)


def _final_body(p_ref, xs_ref, cnt_ref, wz_ref, bz_ref, lz_ref, lzb_ref,
                wh_ref, bh_ref, lh_ref, lhb_ref, w1_ref, b1_ref,
                hh_ref, out_ref):
    deg = 1.0 + cnt_ref[0, :, 0:1] + cnt_ref[1, :, 0:1]
    dinv = lax.rsqrt(deg)
    y = dinv * (p_ref[0] + p_ref[1] + xs_ref[...])
    hi = lax.Precision.HIGHEST
    cz = jnp.dot(y, wz_ref[...], precision=hi,
                 preferred_element_type=jnp.float32) + bz_ref[...]
    ch = jnp.dot(y, wh_ref[...], precision=hi,
                 preferred_element_type=jnp.float32) + bh_ref[...]
    z = jax.nn.sigmoid(jnp.dot(cz, lz_ref[...], precision=hi,
                               preferred_element_type=jnp.float32) + lzb_ref[...])
    ht = jnp.tanh(jnp.dot(ch, lh_ref[...], precision=hi,
                          preferred_element_type=jnp.float32) + lhb_ref[...])
    hh = jnp.maximum((1.0 - z) * ht, 0.0)
    hh_ref[...] = hh
    out_ref[...] = jnp.dot(hh, w1_ref[...], precision=hi,
                           preferred_element_type=jnp.float32) + b1_ref[...]


def _full_spec(shape):
    return pl.BlockSpec(shape, lambda i: tuple(0 for _ in shape))


def _final_call(parts, xs, counts, Wz, bz, Lz0, Lzb, Wh, bh, Lh0, Lhb, W1, b1):
    return pl.pallas_call(
        _final_body,
        grid=(_NTOT // _ROWS,),
        in_specs=[
            pl.BlockSpec((_NC, _ROWS, _F), lambda i: (0, i, 0)),
            pl.BlockSpec((_ROWS, _F), lambda i: (i, 0)),
            pl.BlockSpec((_NC, _ROWS, _CNTW), lambda i: (0, i, 0)),
            _full_spec((_F, _HID)), _full_spec((1, _HID)),
            _full_spec((_HID, _HID)), _full_spec((1, _HID)),
            _full_spec((_F, _HID)), _full_spec((1, _HID)),
            _full_spec((_HID, _HID)), _full_spec((1, _HID)),
            _full_spec((_HID, _LOOKAHEAD)), _full_spec((1, _LOOKAHEAD)),
        ],
        out_specs=[
            pl.BlockSpec((_ROWS, _HID), lambda i: (i, 0)),
            pl.BlockSpec((_ROWS, _LOOKAHEAD), lambda i: (i, 0)),
        ],
        out_shape=[
            jax.ShapeDtypeStruct((_NTOT, _HID), jnp.float32),
            jax.ShapeDtypeStruct((_NTOT, _LOOKAHEAD), jnp.float32),
        ],
    )(parts, xs, counts, Wz, bz, Lz0, Lzb, Wh, bh, Lh0, Lhb, W1, b1)


def kernel(node_data, edge_index, Wz, bz, Lz_W, Lz_b, Wr, br, Lr_W, Lr_b,
           Wh, bh, Lh_W, Lh_b, W1, b1):
    x = node_data[:, :, _T - 1, :].reshape(_NTOT, _F)
    npad_e = _EPAD - _E
    pad_iota = jnp.arange(npad_e, dtype=jnp.int32)
    src = jnp.concatenate([edge_index[0], pad_iota % _NTOT])
    dst = jnp.concatenate([edge_index[1],
                           _NTOT + pad_iota % (_NPAD - _NTOT)])
    src = src.reshape(_NTILE, _NCHUNK, _CH)
    dst = dst.reshape(_NTILE, _NCHUNK, _CH)

    counts = _sc_count(dst)
    xs = _prep_call(x, counts)
    parts = _sc_scatter(xs, src, dst)
    hh, out = _final_call(
        parts, xs, counts,
        Wz, bz.reshape(1, _HID), Lz_W[:_HID], Lz_b.reshape(1, _HID),
        Wh, bh.reshape(1, _HID), Lh_W[:_HID], Lh_b.reshape(1, _HID),
        W1, b1.reshape(1, _LOOKAHEAD))
    return (out.reshape(_B, _N, _LOOKAHEAD), hh.reshape(_B, _N, _HID))


# speedup vs baseline: 1.0324x; 1.0324x over previous
import functools

import jax
import jax.numpy as jnp
from jax import lax
from jax.experimental import pallas as pl
from jax.experimental.pallas import tpu as pltpu
from jax.experimental.pallas import tpu_sc as plsc

_B, _N, _T, _F = 4, 2500, 12, 128
_HID = 128
_LOOKAHEAD = 12
_NTOT = _B * _N
_E = 320000

_NC, _NS = 2, 16
_NTILE = _NC * _NS
_CH = 128
_NCHUNK = 80
_EPAD = _NTILE * _NCHUNK * _CH
_NPAD = 10240
_RPT = _NPAD // _NS
_ZB = 128
_CNTW = 128
_G = 40
_NGRP = _NCHUNK // _G

_mesh = plsc.VectorSubcoreMesh(core_axis_name="c", subcore_axis_name="s")


@functools.partial(
    pl.kernel,
    out_type=jax.ShapeDtypeStruct((_NC, _NPAD, _CNTW), jnp.float32),
    mesh=_mesh,
    scratch_types=[
        pltpu.VMEM((_CH, _CNTW), jnp.float32),
        pltpu.VMEM((_G, _CH), jnp.int32),
        pltpu.VMEM_SHARED((_NPAD, _CNTW), jnp.float32),
        pltpu.SemaphoreType.DMA,
    ],
)
def _sc_count(dst_hbm, out_hbm, buf_v, dig_v, cnt_sh, sem):
    c = lax.axis_index("c")
    s = lax.axis_index("s")
    wid = c * _NS + s

    @pl.loop(0, _CH)
    def _(i):
        for jj in range(_CNTW // 16):
            buf_v[i, pl.ds(jj * 16, 16)] = jnp.zeros((16,), jnp.float32)

    for k in range(_RPT // _ZB):
        pltpu.sync_copy(buf_v, cnt_sh.at[pl.ds(s * _RPT + k * _ZB, _ZB)])

    @pl.loop(0, _CH)
    def _(i):
        for jj in range(_CNTW // 16):
            buf_v[i, pl.ds(jj * 16, 16)] = jnp.full((16,), 1.0, jnp.float32)

    plsc.subcore_barrier()

    for g in range(_NGRP):
        pltpu.sync_copy(dst_hbm.at[wid, pl.ds(g * _G, _G)], dig_v)

        @pl.loop(0, _G, step=4)
        def _(j):
            handles = [
                pltpu.async_copy(buf_v, cnt_sh.at[dig_v.at[j + d]], sem,
                                 add=True)
                for d in range(4)
            ]
            for h in handles:
                h.wait()

    plsc.subcore_barrier()
    pltpu.sync_copy(cnt_sh.at[pl.ds(s * _RPT, _RPT)],
                    out_hbm.at[c, pl.ds(s * _RPT, _RPT)])


@functools.partial(
    pl.kernel,
    out_type=jax.ShapeDtypeStruct((_NC, _NPAD, _F), jnp.float32),
    mesh=_mesh,
    scratch_types=[
        pltpu.VMEM((_CH, _F), jnp.float32),
        pltpu.VMEM((_CH, _F), jnp.float32),
        pltpu.VMEM((_G, _CH), jnp.int32),
        pltpu.VMEM((_G, _CH), jnp.int32),
        pltpu.VMEM_SHARED((_NPAD, _F), jnp.float32),
        pltpu.SemaphoreType.DMA,
        pltpu.SemaphoreType.DMA,
    ],
)
def _sc_scatter(xs_hbm, src_hbm, dst_hbm, out_hbm,
                rows0_v, rows1_v, sig_v, dig_v, acc_sh, sem0, sem1):
    c = lax.axis_index("c")
    s = lax.axis_index("s")
    wid = c * _NS + s

    @pl.loop(0, _ZB)
    def _(i):
        for jj in range(_F // 16):
            rows0_v[i, pl.ds(jj * 16, 16)] = jnp.zeros((16,), jnp.float32)

    for k in range(_RPT // _ZB):
        pltpu.sync_copy(rows0_v, acc_sh.at[pl.ds(s * _RPT + k * _ZB, _ZB)])
    plsc.subcore_barrier()

    for g in range(_NGRP):
        pltpu.sync_copy(src_hbm.at[wid, pl.ds(g * _G, _G)], sig_v)
        pltpu.sync_copy(dst_hbm.at[wid, pl.ds(g * _G, _G)], dig_v)
        pltpu.async_copy(xs_hbm.at[sig_v.at[0]], rows0_v, sem0)

        @pl.loop(0, _G, step=2)
        def _(j):
            pltpu.async_copy(xs_hbm.at[sig_v.at[j + 1]], rows1_v, sem1)
            pltpu.make_async_copy(xs_hbm.at[pl.ds(0, _CH)], rows0_v,
                                  sem0).wait()
            pltpu.sync_copy(rows0_v, acc_sh.at[dig_v.at[j]], add=True)

            @pl.when(j + 2 < _G)
            def _():
                pltpu.async_copy(xs_hbm.at[sig_v.at[j + 2]], rows0_v, sem0)

            pltpu.make_async_copy(xs_hbm.at[pl.ds(0, _CH)], rows1_v,
                                  sem1).wait()
            pltpu.sync_copy(rows1_v, acc_sh.at[dig_v.at[j + 1]], add=True)

    plsc.subcore_barrier()
    pltpu.sync_copy(acc_sh.at[pl.ds(s * _RPT, _RPT)],
                    out_hbm.at[c, pl.ds(s * _RPT, _RPT)])


_ROWS = 1000


def _prep_body(x_ref, cnt_ref, xs_ref):
    deg = 1.0 + cnt_ref[0, :, 0:1] + cnt_ref[1, :, 0:1]
    xs_ref[...] = x_ref[...] * lax.rsqrt(deg)


def _prep_call(x, counts):
    return pl.pallas_call(
        _prep_body,
        grid=(_NTOT // _ROWS,),
        in_specs=[
            pl.BlockSpec((_ROWS, _F), lambda i: (i, 0)),
            pl.BlockSpec((_NC, _ROWS, _CNTW), lambda i: (0, i, 0)),
        ],
        out_specs=pl.BlockSpec((_ROWS, _F), lambda i: (i, 0)),
        out_shape=jax.ShapeDtypeStruct((_NTOT, _F), jnp.float32),
    )(x, counts)


def _final_body(p_ref, xs_ref, cnt_ref, wz_ref, bz_ref, lz_ref, lzb_ref,
                wh_ref, bh_ref, lh_ref, lhb_ref, w1_ref, b1_ref,
                hh_ref, out_ref):
    deg = 1.0 + cnt_ref[0, :, 0:1] + cnt_ref[1, :, 0:1]
    dinv = lax.rsqrt(deg)
    y = dinv * (p_ref[0] + p_ref[1] + xs_ref[...])
    hi = lax.Precision.HIGHEST
    cz = jnp.dot(y, wz_ref[...], precision=hi,
                 preferred_element_type=jnp.float32) + bz_ref[...]
    ch = jnp.dot(y, wh_ref[...], precision=hi,
                 preferred_element_type=jnp.float32) + bh_ref[...]
    z = jax.nn.sigmoid(jnp.dot(cz, lz_ref[...], precision=hi,
                               preferred_element_type=jnp.float32) + lzb_ref[...])
    ht = jnp.tanh(jnp.dot(ch, lh_ref[...], precision=hi,
                          preferred_element_type=jnp.float32) + lhb_ref[...])
    hh = jnp.maximum((1.0 - z) * ht, 0.0)
    hh_ref[...] = hh
    out_ref[...] = jnp.dot(hh, w1_ref[...], precision=hi,
                           preferred_element_type=jnp.float32) + b1_ref[...]


def _full_spec(shape):
    return pl.BlockSpec(shape, lambda i: tuple(0 for _ in shape))


def _final_call(parts, xs, counts, Wz, bz, Lz0, Lzb, Wh, bh, Lh0, Lhb, W1, b1):
    return pl.pallas_call(
        _final_body,
        grid=(_NTOT // _ROWS,),
        in_specs=[
            pl.BlockSpec((_NC, _ROWS, _F), lambda i: (0, i, 0)),
            pl.BlockSpec((_ROWS, _F), lambda i: (i, 0)),
            pl.BlockSpec((_NC, _ROWS, _CNTW), lambda i: (0, i, 0)),
            _full_spec((_F, _HID)), _full_spec((1, _HID)),
            _full_spec((_HID, _HID)), _full_spec((1, _HID)),
            _full_spec((_F, _HID)), _full_spec((1, _HID)),
            _full_spec((_HID, _HID)), _full_spec((1, _HID)),
            _full_spec((_HID, _LOOKAHEAD)), _full_spec((1, _LOOKAHEAD)),
        ],
        out_specs=[
            pl.BlockSpec((_ROWS, _HID), lambda i: (i, 0)),
            pl.BlockSpec((_ROWS, _LOOKAHEAD), lambda i: (i, 0)),
        ],
        out_shape=[
            jax.ShapeDtypeStruct((_NTOT, _HID), jnp.float32),
            jax.ShapeDtypeStruct((_NTOT, _LOOKAHEAD), jnp.float32),
        ],
    )(parts, xs, counts, Wz, bz, Lz0, Lzb, Wh, bh, Lh0, Lhb, W1, b1)


def kernel(node_data, edge_index, Wz, bz, Lz_W, Lz_b, Wr, br, Lr_W, Lr_b,
           Wh, bh, Lh_W, Lh_b, W1, b1):
    x = node_data[:, :, _T - 1, :].reshape(_NTOT, _F)
    npad_e = _EPAD - _E
    pad_iota = jnp.arange(npad_e, dtype=jnp.int32)
    src = jnp.concatenate([edge_index[0], pad_iota % _NTOT])
    dst = jnp.concatenate([edge_index[1],
                           _NTOT + pad_iota % (_NPAD - _NTOT)])
    src = src.reshape(_NTILE, _NCHUNK, _CH)
    dst = dst.reshape(_NTILE, _NCHUNK, _CH)

    counts = _sc_count(dst)
    xs = _prep_call(x, counts)
    parts = _sc_scatter(xs, src, dst)
    hh, out = _final_call(
        parts, xs, counts,
        Wz, bz.reshape(1, _HID), Lz_W[:_HID], Lz_b.reshape(1, _HID),
        Wh, bh.reshape(1, _HID), Lh_W[:_HID], Lh_b.reshape(1, _HID),
        W1, b1.reshape(1, _LOOKAHEAD))
    return (out.reshape(_B, _N, _LOOKAHEAD), hh.reshape(_B, _N, _HID))
